# Initial kernel scaffold; baseline (speedup 1.0000x reference)
import jax
import jax.numpy as jnp
from jax.experimental import pallas as pl


def kernel(user0, item_i0, ratings, embed_user, embed_item, edge_u, edge_i, edge_val, d_i, d_j, W_add, W1, b1, W2, b2, ub_tab, ib_tab, avg_rating):
    raise NotImplementedError("write your pallas kernel here")


# speedup vs baseline: 6.4096x; 6.4096x over previous
import functools

import jax
import jax.numpy as jnp
from jax import lax
from jax.experimental import pallas as pl
from jax.experimental.pallas import tpu as pltpu
from jax.experimental.pallas import tpu_sc as plsc

N = 50000
D = 32
E = 1600000
B = 16384
LAM = 0.001
NC = 2
NS = 16
CH = 128
EPT = E // NS
FC = EPT // CH
ETAIL = EPT - FC * CH
NODE_CHUNKS = N // CH
NTAIL = N - NODE_CHUNKS * CH
RB_ITERS = NODE_CHUNKS // NS + 1

_mesh = plsc.VectorSubcoreMesh(
    core_axis_name="c", subcore_axis_name="s", num_cores=NC, num_subcores=NS)

_f32 = jnp.float32
_i32 = jnp.int32


def _zero_rows(buf, n):
  z = jnp.zeros((16,), _f32)

  @pl.loop(0, n)
  def _(e):
    buf[e, pl.ds(0, 16)] = z
    buf[e, pl.ds(16, 16)] = z


def _zero_acc(acc, zbuf, s):
  _zero_rows(zbuf, CH)

  @pl.loop(0, RB_ITERS)
  def _(ti):
    t = ti * NS + s

    @pl.when(t < NODE_CHUNKS)
    def _():
      pltpu.sync_copy(zbuf, acc.at[pl.ds(t * CH, CH)])

    @pl.when(t == NODE_CHUNKS)
    def _():
      pltpu.sync_copy(zbuf.at[pl.ds(0, NTAIL)], acc.at[pl.ds(NODE_CHUNKS * CH, NTAIL)])


def _edge_phase(s, sidx_hbm, gidx_hbm, ev_hbm, tab_hbm, acc,
                gidx, sidx, valb, rows, sem):
  base0 = s * EPT

  def do_chunk(base, n):
    pltpu.sync_copy(gidx_hbm.at[pl.ds(base, n)], gidx.at[pl.ds(0, n)])
    pltpu.sync_copy(sidx_hbm.at[pl.ds(base, n)], sidx.at[0, pl.ds(0, n)])
    pltpu.sync_copy(ev_hbm.at[pl.ds(base, n)], valb.at[pl.ds(0, n)])
    pltpu.async_copy(tab_hbm.at[gidx], rows, sem).wait()

    @pl.loop(0, CH // 16)
    def _(g):
      vv = valb[pl.ds(g * 16, 16)]
      for j in range(16):
        v = vv[j]
        e = g * 16 + j
        rows[e, pl.ds(0, 16)] = rows[e, pl.ds(0, 16)] * v
        rows[e, pl.ds(16, 16)] = rows[e, pl.ds(16, 16)] * v

    pltpu.sync_copy(rows, acc.at[sidx.at[0]], add=True)

  @pl.loop(0, FC)
  def _(ci):
    do_chunk(base0 + ci * CH, CH)

  zf = jnp.zeros((16,), _f32)
  zi = jnp.zeros((16,), _i32)
  for k in range(ETAIL, CH, 16):
    valb[pl.ds(k, 16)] = zf
    gidx[pl.ds(k, 16)] = zi
    sidx[0, pl.ds(k, 16)] = zi
  do_chunk(base0 + FC * CH, ETAIL)


def _readback_l1(s, acc, base_hbm, d_hbm, out_hbm, abuf, bbuf, dbuf, obuf):
  def do_chunk(r0, n):
    pltpu.sync_copy(acc.at[pl.ds(r0, n)], abuf.at[pl.ds(0, n)])
    pltpu.sync_copy(base_hbm.at[pl.ds(r0, n)], bbuf.at[pl.ds(0, n)])
    pltpu.sync_copy(d_hbm.at[pl.ds(r0, n)], dbuf.at[pl.ds(0, n)])

    @pl.loop(0, n // 16)
    def _(g):
      dv = dbuf[pl.ds(g * 16, 16)]
      for j in range(16):
        d = dv[j]
        e = g * 16 + j
        for h in (0, 16):
          obuf[e, pl.ds(h, 16)] = jnp.maximum(
              abuf[e, pl.ds(h, 16)] + bbuf[e, pl.ds(h, 16)] * d, 0.0)

    pltpu.sync_copy(obuf.at[pl.ds(0, n)], out_hbm.at[pl.ds(r0, n)])

  @pl.loop(0, RB_ITERS)
  def _(ti):
    t = ti * NS + s

    @pl.when(t < NODE_CHUNKS)
    def _():
      do_chunk(t * CH, CH)

    @pl.when(t == NODE_CHUNKS)
    def _():
      do_chunk(NODE_CHUNKS * CH, NTAIL)


def _readback_l2(s, acc, g1_hbm, emb_hbm, d_hbm, out_hbm,
                 abuf, bbuf, ebuf, dbuf, obuf, wbuf, ssqv):
  wv = wbuf[...]
  w0 = wv[0]
  w1 = wv[1]
  w2 = wv[2]
  ssqv[...] = jnp.zeros((16,), _f32)

  def do_chunk(r0, n):
    pltpu.sync_copy(acc.at[pl.ds(r0, n)], abuf.at[pl.ds(0, n)])
    pltpu.sync_copy(g1_hbm.at[pl.ds(r0, n)], bbuf.at[pl.ds(0, n)])
    pltpu.sync_copy(emb_hbm.at[pl.ds(r0, n)], ebuf.at[pl.ds(0, n)])
    pltpu.sync_copy(d_hbm.at[pl.ds(r0, n)], dbuf.at[pl.ds(0, n)])

    @pl.loop(0, n // 16)
    def _(g):
      dv = dbuf[pl.ds(g * 16, 16)]
      acc_sq = jnp.zeros((16,), _f32)
      for j in range(16):
        d = dv[j]
        e = g * 16 + j
        for h in (0, 16):
          g1 = bbuf[e, pl.ds(h, 16)]
          g2 = jnp.maximum(abuf[e, pl.ds(h, 16)] + g1 * d, 0.0)
          o = w0 * ebuf[e, pl.ds(h, 16)] + w1 * g1 + w2 * g2
          obuf[e, pl.ds(h, 16)] = o
          acc_sq = acc_sq + o * o
      ssqv[...] = ssqv[...] + acc_sq

    pltpu.sync_copy(obuf.at[pl.ds(0, n)], out_hbm.at[pl.ds(r0, n)])

  @pl.loop(0, RB_ITERS)
  def _(ti):
    t = ti * NS + s

    @pl.when(t < NODE_CHUNKS)
    def _():
      do_chunk(t * CH, CH)

    @pl.when(t == NODE_CHUNKS)
    def _():
      do_chunk(NODE_CHUNKS * CH, NTAIL)


def _layer1_body(eu, ei, ev, tu, ti, du, dv, out_u, out_i,
                 acc, gidx, sidx, valb, rows, abuf, bbuf, dbuf, obuf, sem):
  c = lax.axis_index("c")
  s = lax.axis_index("s")
  _zero_acc(acc, obuf, s)
  plsc.subcore_barrier()

  @pl.when(c == 0)
  def _():
    _edge_phase(s, eu, ei, ev, ti, acc, gidx, sidx, valb, rows, sem)

  @pl.when(c == 1)
  def _():
    _edge_phase(s, ei, eu, ev, tu, acc, gidx, sidx, valb, rows, sem)

  plsc.subcore_barrier()

  @pl.when(c == 0)
  def _():
    _readback_l1(s, acc, tu, du, out_u, abuf, bbuf, dbuf, obuf)

  @pl.when(c == 1)
  def _():
    _readback_l1(s, acc, ti, dv, out_i, abuf, bbuf, dbuf, obuf)


def _layer2_body(eu, ei, ev, g1u, g1i, embu, embi, du, dv, wpad,
                 out_u, out_i, parts,
                 acc, gidx, sidx, valb, rows, abuf, bbuf, ebuf, dbuf, obuf,
                 wbuf, ssqv, sem):
  c = lax.axis_index("c")
  s = lax.axis_index("s")
  pltpu.sync_copy(wpad, wbuf)
  _zero_acc(acc, obuf, s)
  plsc.subcore_barrier()

  @pl.when(c == 0)
  def _():
    _edge_phase(s, eu, ei, ev, g1i, acc, gidx, sidx, valb, rows, sem)

  @pl.when(c == 1)
  def _():
    _edge_phase(s, ei, eu, ev, g1u, acc, gidx, sidx, valb, rows, sem)

  plsc.subcore_barrier()

  @pl.when(c == 0)
  def _():
    _readback_l2(s, acc, g1u, embu, du, out_u,
                 abuf, bbuf, ebuf, dbuf, obuf, wbuf, ssqv)

  @pl.when(c == 1)
  def _():
    _readback_l2(s, acc, g1i, embi, dv, out_i,
                 abuf, bbuf, ebuf, dbuf, obuf, wbuf, ssqv)

  pltpu.sync_copy(ssqv, parts.at[c, s])


def _gather_body(u0, i0, gu, gi, ub, ib, ug, ig, ubg, ibg,
                 idxb, rowsb, sbuf, sem):
  c = lax.axis_index("c")
  s = lax.axis_index("s")

  def side(idx_hbm, tab, btab, rows_out, b_out):
    @pl.loop(0, B // (NS * CH))
    def _(j):
      base = (s * (B // (NS * CH)) + j) * CH
      pltpu.sync_copy(idx_hbm.at[pl.ds(base, CH)], idxb)
      pltpu.async_copy(tab.at[idxb], rowsb, sem).wait()
      pltpu.sync_copy(rowsb, rows_out.at[pl.ds(base, CH)])
      pltpu.async_copy(btab.at[idxb], sbuf, sem).wait()
      pltpu.sync_copy(sbuf, b_out.at[pl.ds(base, CH)])

  @pl.when(c == 0)
  def _():
    side(u0, gu, ub, ug, ubg)

  @pl.when(c == 1)
  def _():
    side(i0, gi, ib, ig, ibg)


_sc_params = pltpu.CompilerParams(use_tc_tiling_on_sc=False)

_layer1 = pl.kernel(
    _layer1_body,
    out_type=[jax.ShapeDtypeStruct((N, D), _f32)] * 2,
    mesh=_mesh,
    compiler_params=_sc_params,
    scratch_types=[
        pltpu.VMEM_SHARED((N, D), _f32),
        pltpu.VMEM((CH,), _i32),
        pltpu.VMEM((1, CH), _i32),
        pltpu.VMEM((CH,), _f32),
        pltpu.VMEM((CH, D), _f32),
        pltpu.VMEM((CH, D), _f32),
        pltpu.VMEM((CH, D), _f32),
        pltpu.VMEM((CH,), _f32),
        pltpu.VMEM((CH, D), _f32),
        pltpu.SemaphoreType.DMA,
    ],
)

_layer2 = pl.kernel(
    _layer2_body,
    out_type=[
        jax.ShapeDtypeStruct((N, D), _f32),
        jax.ShapeDtypeStruct((N, D), _f32),
        jax.ShapeDtypeStruct((NC, NS, 16), _f32),
    ],
    mesh=_mesh,
    compiler_params=_sc_params,
    scratch_types=[
        pltpu.VMEM_SHARED((N, D), _f32),
        pltpu.VMEM((CH,), _i32),
        pltpu.VMEM((1, CH), _i32),
        pltpu.VMEM((CH,), _f32),
        pltpu.VMEM((CH, D), _f32),
        pltpu.VMEM((CH, D), _f32),
        pltpu.VMEM((CH, D), _f32),
        pltpu.VMEM((CH, D), _f32),
        pltpu.VMEM((CH,), _f32),
        pltpu.VMEM((CH, D), _f32),
        pltpu.VMEM((16,), _f32),
        pltpu.VMEM((16,), _f32),
        pltpu.SemaphoreType.DMA,
    ],
)

_gatherk = pl.kernel(
    _gather_body,
    out_type=[
        jax.ShapeDtypeStruct((B, D), _f32),
        jax.ShapeDtypeStruct((B, D), _f32),
        jax.ShapeDtypeStruct((B, 1), _f32),
        jax.ShapeDtypeStruct((B, 1), _f32),
    ],
    mesh=_mesh,
    compiler_params=_sc_params,
    scratch_types=[
        pltpu.VMEM((CH,), _i32),
        pltpu.VMEM((CH, D), _f32),
        pltpu.VMEM((CH, 1), _f32),
        pltpu.SemaphoreType.DMA,
    ],
)

_BLK = 2048
_NB = B // _BLK


def _leaky(x):
  return jnp.where(x >= 0, x, 0.1 * x)


def _tc_body(ug, ig, ubg, ibg, rat, w1t, b1r, w2t, b2r, avg, parts,
             loss_ref, loss2_ref):
  i = pl.program_id(0)

  def filt(x):
    h = _leaky(jnp.dot(x, w1t[...], preferred_element_type=_f32) + b1r[...])
    return _leaky(jnp.dot(h, w2t[...], preferred_element_type=_f32) + b2r[...])

  u = filt(ug[...])
  v = filt(ig[...])
  pred = (jnp.sum(u * v, axis=1, keepdims=True) + ubg[...] + ibg[...]
          + avg[0, 0])
  sq = jnp.sum((pred - rat[...]) ** 2)

  @pl.when(i == 0)
  def _():
    loss2_ref[...] = jnp.zeros((1, 1), _f32)

  loss2_ref[...] = loss2_ref[...] + sq

  @pl.when(i == _NB - 1)
  def _():
    l2 = LAM * jnp.sum(parts[...]) / (N * D)
    m = loss2_ref[...] / B
    loss2_ref[...] = m
    loss_ref[...] = m + l2


_tck = pl.pallas_call(
    _tc_body,
    grid=(_NB,),
    in_specs=[
        pl.BlockSpec((_BLK, D), lambda i: (i, 0)),
        pl.BlockSpec((_BLK, D), lambda i: (i, 0)),
        pl.BlockSpec((_BLK, 1), lambda i: (i, 0)),
        pl.BlockSpec((_BLK, 1), lambda i: (i, 0)),
        pl.BlockSpec((_BLK, 1), lambda i: (i, 0)),
        pl.BlockSpec((D, 2 * D), lambda i: (0, 0)),
        pl.BlockSpec((1, 2 * D), lambda i: (0, 0)),
        pl.BlockSpec((2 * D, D), lambda i: (0, 0)),
        pl.BlockSpec((1, D), lambda i: (0, 0)),
        pl.BlockSpec((1, 1), lambda i: (0, 0)),
        pl.BlockSpec((NC * NS, 16), lambda i: (0, 0)),
    ],
    out_specs=[
        pl.BlockSpec((1, 1), lambda i: (0, 0)),
        pl.BlockSpec((1, 1), lambda i: (0, 0)),
    ],
    out_shape=[
        jax.ShapeDtypeStruct((1, 1), _f32),
        jax.ShapeDtypeStruct((1, 1), _f32),
    ],
)


@jax.jit
def kernel(user0, item_i0, ratings, embed_user, embed_item, edge_u, edge_i,
           edge_val, d_i, d_j, W_add, W1, b1, W2, b2, ub_tab, ib_tab,
           avg_rating):
  eu = edge_u.astype(_i32)
  ei = edge_i.astype(_i32)
  u0 = user0.astype(_i32)
  it0 = item_i0.astype(_i32)
  ev = edge_val.astype(_f32)

  dif = jnp.reshape(d_i, (N,)).astype(_f32)
  djf = jnp.reshape(d_j, (N,)).astype(_f32)
  g1u, g1i = _layer1(eu, ei, ev, embed_user, embed_item, dif, djf)
  wpad = jnp.zeros((16,), _f32).at[0:3].set(W_add[0].astype(_f32))
  gu, gi, parts = _layer2(eu, ei, ev, g1u, g1i, embed_user, embed_item,
                          dif, djf, wpad)
  ug, ig, ubg, ibg = _gatherk(u0, it0, gu, gi, ub_tab, ib_tab)
  loss, loss2 = _tck(ug, ig, ubg, ibg, jnp.reshape(ratings, (B, 1)),
                     W1.T, jnp.reshape(b1, (1, 2 * D)),
                     W2.T, jnp.reshape(b2, (1, D)),
                     jnp.reshape(avg_rating, (1, 1)).astype(_f32),
                     jnp.reshape(parts, (NC * NS, 16)))
  return (loss[0, 0], loss2[0, 0])


# speedup vs baseline: 21.3971x; 3.3383x over previous
import functools

import jax
import jax.numpy as jnp
from jax import lax
from jax.experimental import pallas as pl
from jax.experimental.pallas import tpu as pltpu
from jax.experimental.pallas import tpu_sc as plsc

N = 50000
D = 32
E = 1600000
B = 16384
LAM = 0.001
NC = 2
NS = 16
CH = 128
ECH = E // CH
K = 10
NB = 5
LA = 3
NBLK = ECH // K
EB_ITERS = (NBLK + NS - 1) // NS
NODE_CHUNKS = N // CH
NTAIL = N - NODE_CHUNKS * CH
RB_ITERS = NODE_CHUNKS // NS + 1

_mesh = plsc.VectorSubcoreMesh(
    core_axis_name="c", subcore_axis_name="s", num_cores=NC, num_subcores=NS)

_f32 = jnp.float32
_i32 = jnp.int32


def _zero_rows(buf, n):
  z = jnp.zeros((16,), _f32)

  @pl.loop(0, n)
  def _(e):
    buf[e, pl.ds(0, 16)] = z
    buf[e, pl.ds(16, 16)] = z


def _zero_acc(acc, zbuf, s):
  _zero_rows(zbuf, CH)

  @pl.loop(0, RB_ITERS)
  def _(ti):
    t = ti * NS + s

    @pl.when(t < NODE_CHUNKS)
    def _():
      pltpu.sync_copy(zbuf, acc.at[pl.ds(t * CH, CH)])

    @pl.when(t == NODE_CHUNKS)
    def _():
      pltpu.sync_copy(zbuf.at[pl.ds(0, NTAIL)], acc.at[pl.ds(NODE_CHUNKS * CH, NTAIL)])


def _edge_phase(s, sidx_hbm, gidx_hbm, ev_hbm, tab_hbm, acc,
                sidxb, gidxb, valb, rows, gsems, ssems):
  @pl.loop(0, EB_ITERS)
  def _(bi):
    t = bi * NS + s

    @pl.when(t < NBLK)
    def _():
      b0 = t * K
      pltpu.sync_copy(sidx_hbm.at[pl.ds(b0, K)], sidxb)
      pltpu.sync_copy(gidx_hbm.at[pl.ds(b0, K)], gidxb)
      pltpu.sync_copy(ev_hbm.at[pl.ds(b0, K)], valb)

      gd = {}
      sd = {}
      s_waited = [False] * K

      def fire_gather(k):
        gd[k] = pltpu.async_copy(
            tab_hbm.at[gidxb.at[k]], rows[k % NB], gsems[k % NB])

      for k in range(min(LA, K)):
        fire_gather(k)

      for k in range(K):
        gd[k].wait()

        @pl.loop(0, CH // 16)
        def _(g):
          vv = valb[k, pl.ds(g * 16, 16)]
          for j in range(16):
            v = vv[j]
            e = g * 16 + j
            rows[k % NB][e, pl.ds(0, 16)] = rows[k % NB][e, pl.ds(0, 16)] * v
            rows[k % NB][e, pl.ds(16, 16)] = rows[k % NB][e, pl.ds(16, 16)] * v

        sd[k] = pltpu.async_copy(
            rows[k % NB], acc.at[sidxb.at[k]], ssems[k % NB], add=True)
        nk = k + LA
        if nk < K:
          if nk >= NB:
            sd[nk - NB].wait()
            s_waited[nk - NB] = True
          fire_gather(nk)

      for k in range(K):
        if not s_waited[k]:
          sd[k].wait()


def _readback_l1(s, acc, base_hbm, d_hbm, out_hbm, abuf, bbuf, dbuf, obuf):
  def do_chunk(r0, n):
    pltpu.sync_copy(acc.at[pl.ds(r0, n)], abuf.at[pl.ds(0, n)])
    pltpu.sync_copy(base_hbm.at[pl.ds(r0, n)], bbuf.at[pl.ds(0, n)])
    pltpu.sync_copy(d_hbm.at[pl.ds(r0, n)], dbuf.at[pl.ds(0, n)])

    @pl.loop(0, n // 16)
    def _(g):
      dv = dbuf[pl.ds(g * 16, 16)]
      for j in range(16):
        d = dv[j]
        e = g * 16 + j
        for h in (0, 16):
          obuf[e, pl.ds(h, 16)] = jnp.maximum(
              abuf[e, pl.ds(h, 16)] + bbuf[e, pl.ds(h, 16)] * d, 0.0)

    pltpu.sync_copy(obuf.at[pl.ds(0, n)], out_hbm.at[pl.ds(r0, n)])

  @pl.loop(0, RB_ITERS)
  def _(ti):
    t = ti * NS + s

    @pl.when(t < NODE_CHUNKS)
    def _():
      do_chunk(t * CH, CH)

    @pl.when(t == NODE_CHUNKS)
    def _():
      do_chunk(NODE_CHUNKS * CH, NTAIL)


def _readback_l2(s, acc, g1_hbm, emb_hbm, d_hbm, out_hbm,
                 abuf, bbuf, ebuf, dbuf, obuf, wbuf, ssqv):
  wv = wbuf[...]
  w0 = wv[0]
  w1 = wv[1]
  w2 = wv[2]
  ssqv[...] = jnp.zeros((16,), _f32)

  def do_chunk(r0, n):
    pltpu.sync_copy(acc.at[pl.ds(r0, n)], abuf.at[pl.ds(0, n)])
    pltpu.sync_copy(g1_hbm.at[pl.ds(r0, n)], bbuf.at[pl.ds(0, n)])
    pltpu.sync_copy(emb_hbm.at[pl.ds(r0, n)], ebuf.at[pl.ds(0, n)])
    pltpu.sync_copy(d_hbm.at[pl.ds(r0, n)], dbuf.at[pl.ds(0, n)])

    @pl.loop(0, n // 16)
    def _(g):
      dv = dbuf[pl.ds(g * 16, 16)]
      acc_sq = jnp.zeros((16,), _f32)
      for j in range(16):
        d = dv[j]
        e = g * 16 + j
        for h in (0, 16):
          g1 = bbuf[e, pl.ds(h, 16)]
          g2 = jnp.maximum(abuf[e, pl.ds(h, 16)] + g1 * d, 0.0)
          o = w0 * ebuf[e, pl.ds(h, 16)] + w1 * g1 + w2 * g2
          obuf[e, pl.ds(h, 16)] = o
          acc_sq = acc_sq + o * o
      ssqv[...] = ssqv[...] + acc_sq

    pltpu.sync_copy(obuf.at[pl.ds(0, n)], out_hbm.at[pl.ds(r0, n)])

  @pl.loop(0, RB_ITERS)
  def _(ti):
    t = ti * NS + s

    @pl.when(t < NODE_CHUNKS)
    def _():
      do_chunk(t * CH, CH)

    @pl.when(t == NODE_CHUNKS)
    def _():
      do_chunk(NODE_CHUNKS * CH, NTAIL)


def _layer1_body(eu, ei, ev, tu, ti, du, dv, out_u, out_i,
                 acc, sidxb, gidxb, valb, r0, r1, r2, r3, r4, dbuf,
                 g0, g1, g2, g3, g4, s0, s1, s2, s3, s4):
  rows = [r0, r1, r2, r3, r4]
  gsems = [g0, g1, g2, g3, g4]
  ssems = [s0, s1, s2, s3, s4]
  abuf, bbuf, obuf = r0, r1, r2
  c = lax.axis_index("c")
  s = lax.axis_index("s")
  _zero_acc(acc, obuf, s)
  plsc.subcore_barrier()

  @pl.when(c == 0)
  def _():
    _edge_phase(s, eu, ei, ev, ti, acc, sidxb, gidxb, valb, rows, gsems, ssems)

  @pl.when(c == 1)
  def _():
    _edge_phase(s, ei, eu, ev, tu, acc, sidxb, gidxb, valb, rows, gsems, ssems)

  plsc.subcore_barrier()

  @pl.when(c == 0)
  def _():
    _readback_l1(s, acc, tu, du, out_u, abuf, bbuf, dbuf, obuf)

  @pl.when(c == 1)
  def _():
    _readback_l1(s, acc, ti, dv, out_i, abuf, bbuf, dbuf, obuf)


def _layer2_body(eu, ei, ev, g1u, g1i, embu, embi, du, dv, wpad,
                 out_u, out_i, parts,
                 acc, sidxb, gidxb, valb, r0, r1, r2, r3, r4, dbuf, wbuf, ssqv,
                 g0, g1_, g2, g3, g4, s0, s1, s2, s3, s4):
  rows = [r0, r1, r2, r3, r4]
  gsems = [g0, g1_, g2, g3, g4]
  ssems = [s0, s1, s2, s3, s4]
  abuf, bbuf, ebuf, obuf = r0, r1, r2, r3
  c = lax.axis_index("c")
  s = lax.axis_index("s")
  pltpu.sync_copy(wpad, wbuf)
  _zero_acc(acc, obuf, s)
  plsc.subcore_barrier()

  @pl.when(c == 0)
  def _():
    _edge_phase(s, eu, ei, ev, g1i, acc, sidxb, gidxb, valb, rows, gsems, ssems)

  @pl.when(c == 1)
  def _():
    _edge_phase(s, ei, eu, ev, g1u, acc, sidxb, gidxb, valb, rows, gsems, ssems)

  plsc.subcore_barrier()

  @pl.when(c == 0)
  def _():
    _readback_l2(s, acc, g1u, embu, du, out_u,
                 abuf, bbuf, ebuf, dbuf, obuf, wbuf, ssqv)

  @pl.when(c == 1)
  def _():
    _readback_l2(s, acc, g1i, embi, dv, out_i,
                 abuf, bbuf, ebuf, dbuf, obuf, wbuf, ssqv)

  pltpu.sync_copy(ssqv, parts.at[c, s])


def _gather_body(u0, i0, gu, gi, ub, ib, ug, ig, ubg, ibg,
                 idxb, rowsb, sbuf, sem):
  c = lax.axis_index("c")
  s = lax.axis_index("s")

  def side(idx_hbm, tab, btab, rows_out, b_out):
    @pl.loop(0, B // (NS * CH))
    def _(j):
      base = (s * (B // (NS * CH)) + j) * CH
      pltpu.sync_copy(idx_hbm.at[pl.ds(base, CH)], idxb)
      pltpu.async_copy(tab.at[idxb], rowsb, sem).wait()
      pltpu.sync_copy(rowsb, rows_out.at[pl.ds(base, CH)])
      pltpu.async_copy(btab.at[idxb], sbuf, sem).wait()
      pltpu.sync_copy(sbuf, b_out.at[pl.ds(base, CH)])

  @pl.when(c == 0)
  def _():
    side(u0, gu, ub, ug, ubg)

  @pl.when(c == 1)
  def _():
    side(i0, gi, ib, ig, ibg)


_sc_params = pltpu.CompilerParams(use_tc_tiling_on_sc=False)

_layer1 = pl.kernel(
    _layer1_body,
    out_type=[jax.ShapeDtypeStruct((N, D), _f32)] * 2,
    mesh=_mesh,
    compiler_params=_sc_params,
    scratch_types=(
        [
            pltpu.VMEM_SHARED((N, D), _f32),
            pltpu.VMEM((K, CH), _i32),
            pltpu.VMEM((K, CH), _i32),
            pltpu.VMEM((K, CH), _f32),
        ]
        + [pltpu.VMEM((CH, D), _f32)] * NB
        + [pltpu.VMEM((CH,), _f32)]
        + [pltpu.SemaphoreType.DMA] * (2 * NB)
    ),
)

_layer2 = pl.kernel(
    _layer2_body,
    out_type=[
        jax.ShapeDtypeStruct((N, D), _f32),
        jax.ShapeDtypeStruct((N, D), _f32),
        jax.ShapeDtypeStruct((NC, NS, 16), _f32),
    ],
    mesh=_mesh,
    compiler_params=_sc_params,
    scratch_types=(
        [
            pltpu.VMEM_SHARED((N, D), _f32),
            pltpu.VMEM((K, CH), _i32),
            pltpu.VMEM((K, CH), _i32),
            pltpu.VMEM((K, CH), _f32),
        ]
        + [pltpu.VMEM((CH, D), _f32)] * NB
        + [
            pltpu.VMEM((CH,), _f32),
            pltpu.VMEM((16,), _f32),
            pltpu.VMEM((16,), _f32),
        ]
        + [pltpu.SemaphoreType.DMA] * (2 * NB)
    ),
)

_gatherk = pl.kernel(
    _gather_body,
    out_type=[
        jax.ShapeDtypeStruct((B, D), _f32),
        jax.ShapeDtypeStruct((B, D), _f32),
        jax.ShapeDtypeStruct((B, 1), _f32),
        jax.ShapeDtypeStruct((B, 1), _f32),
    ],
    mesh=_mesh,
    compiler_params=_sc_params,
    scratch_types=[
        pltpu.VMEM((CH,), _i32),
        pltpu.VMEM((CH, D), _f32),
        pltpu.VMEM((CH, 1), _f32),
        pltpu.SemaphoreType.DMA,
    ],
)

_BLK = 2048
_NB = B // _BLK


def _leaky(x):
  return jnp.where(x >= 0, x, 0.1 * x)


def _tc_body(ug, ig, ubg, ibg, rat, w1t, b1r, w2t, b2r, avg, parts,
             loss_ref, loss2_ref):
  i = pl.program_id(0)

  def filt(x):
    h = _leaky(jnp.dot(x, w1t[...], preferred_element_type=_f32) + b1r[...])
    return _leaky(jnp.dot(h, w2t[...], preferred_element_type=_f32) + b2r[...])

  u = filt(ug[...])
  v = filt(ig[...])
  pred = (jnp.sum(u * v, axis=1, keepdims=True) + ubg[...] + ibg[...]
          + avg[0, 0])
  sq = jnp.sum((pred - rat[...]) ** 2)

  @pl.when(i == 0)
  def _():
    loss2_ref[...] = jnp.zeros((1, 1), _f32)

  loss2_ref[...] = loss2_ref[...] + sq

  @pl.when(i == _NB - 1)
  def _():
    l2 = LAM * jnp.sum(parts[...]) / (N * D)
    m = loss2_ref[...] / B
    loss2_ref[...] = m
    loss_ref[...] = m + l2


_tck = pl.pallas_call(
    _tc_body,
    grid=(_NB,),
    in_specs=[
        pl.BlockSpec((_BLK, D), lambda i: (i, 0)),
        pl.BlockSpec((_BLK, D), lambda i: (i, 0)),
        pl.BlockSpec((_BLK, 1), lambda i: (i, 0)),
        pl.BlockSpec((_BLK, 1), lambda i: (i, 0)),
        pl.BlockSpec((_BLK, 1), lambda i: (i, 0)),
        pl.BlockSpec((D, 2 * D), lambda i: (0, 0)),
        pl.BlockSpec((1, 2 * D), lambda i: (0, 0)),
        pl.BlockSpec((2 * D, D), lambda i: (0, 0)),
        pl.BlockSpec((1, D), lambda i: (0, 0)),
        pl.BlockSpec((1, 1), lambda i: (0, 0)),
        pl.BlockSpec((NC * NS, 16), lambda i: (0, 0)),
    ],
    out_specs=[
        pl.BlockSpec((1, 1), lambda i: (0, 0)),
        pl.BlockSpec((1, 1), lambda i: (0, 0)),
    ],
    out_shape=[
        jax.ShapeDtypeStruct((1, 1), _f32),
        jax.ShapeDtypeStruct((1, 1), _f32),
    ],
)


@jax.jit
def kernel(user0, item_i0, ratings, embed_user, embed_item, edge_u, edge_i,
           edge_val, d_i, d_j, W_add, W1, b1, W2, b2, ub_tab, ib_tab,
           avg_rating):
  eu = edge_u.astype(_i32)
  ei = edge_i.astype(_i32)
  u0 = user0.astype(_i32)
  it0 = item_i0.astype(_i32)
  ev = edge_val.astype(_f32)

  eu2 = jnp.reshape(eu, (ECH, CH))
  ei2 = jnp.reshape(ei, (ECH, CH))
  ev2 = jnp.reshape(ev, (ECH, CH))
  dif = jnp.reshape(d_i, (N,)).astype(_f32)
  djf = jnp.reshape(d_j, (N,)).astype(_f32)
  g1u, g1i = _layer1(eu2, ei2, ev2, embed_user, embed_item, dif, djf)
  wpad = jnp.zeros((16,), _f32).at[0:3].set(W_add[0].astype(_f32))
  gu, gi, parts = _layer2(eu2, ei2, ev2, g1u, g1i, embed_user, embed_item,
                          dif, djf, wpad)
  ug, ig, ubg, ibg = _gatherk(u0, it0, gu, gi, ub_tab, ib_tab)
  loss, loss2 = _tck(ug, ig, ubg, ibg, jnp.reshape(ratings, (B, 1)),
                     W1.T, jnp.reshape(b1, (1, 2 * D)),
                     W2.T, jnp.reshape(b2, (1, D)),
                     jnp.reshape(avg_rating, (1, 1)).astype(_f32),
                     jnp.reshape(parts, (NC * NS, 16)))
  return (loss[0, 0], loss2[0, 0])


# speedup vs baseline: 27.7889x; 1.2987x over previous
import jax
import jax.numpy as jnp
from jax import lax
from jax.experimental import pallas as pl
from jax.experimental.pallas import tpu as pltpu
from jax.experimental.pallas import tpu_sc as plsc

N = 50000
D = 32
E = 1600000
B = 16384
LAM = 0.001
NC = 2
NS = 16
CH = 128
ECH = E // CH
K = 10
NB = 5
LA = 3
NBLK = ECH // K
NSLOT = ((NBLK + NS - 1) // NS + 1) // 2 * 2
NODE_CHUNKS = N // CH
NTAIL = N - NODE_CHUNKS * CH
RB_ITERS = NODE_CHUNKS // NS + 1
BG_CH = B // NS // CH

_mesh = plsc.VectorSubcoreMesh(
    core_axis_name="c", subcore_axis_name="s", num_cores=NC, num_subcores=NS)

_f32 = jnp.float32
_i32 = jnp.int32


def _zero_rows(buf, n):
  z = jnp.zeros((16,), _f32)

  @pl.loop(0, n)
  def _(e):
    buf[e, pl.ds(0, 16)] = z
    buf[e, pl.ds(16, 16)] = z


def _zero_acc(acc, zbuf, s, sems):
  _zero_rows(zbuf, CH)
  nw = len(sems)

  def fires(ti):
    t = ti * NS + s

    @pl.when(t < NODE_CHUNKS)
    def _():
      pltpu.async_copy(zbuf, acc.at[pl.ds(t * CH, CH)], sems[ti % nw])

    @pl.when(t == NODE_CHUNKS)
    def _():
      pltpu.async_copy(zbuf.at[pl.ds(0, NTAIL)],
                       acc.at[pl.ds(NODE_CHUNKS * CH, NTAIL)], sems[ti % nw])

  def waits(ti):
    t = ti * NS + s

    @pl.when(t < NODE_CHUNKS)
    def _():
      pltpu.make_async_copy(zbuf, acc.at[pl.ds(t * CH, CH)],
                            sems[ti % nw]).wait()

    @pl.when(t == NODE_CHUNKS)
    def _():
      pltpu.make_async_copy(zbuf.at[pl.ds(0, NTAIL)],
                            acc.at[pl.ds(NODE_CHUNKS * CH, NTAIL)],
                            sems[ti % nw]).wait()

  for w0 in range(0, RB_ITERS, 5):
    wave = range(w0, min(w0 + 5, RB_ITERS))
    for ti in wave:
      fires(ti)
    for ti in wave:
      waits(ti)


def _edge_phase(s, sidx_hbm, gidx_hbm, ev_hbm, tab_hbm, acc,
                ibufs, rows, gsems, ssems, isems):
  def fire_idx(t, p):
    b0 = t * K
    sb, gb, vb = ibufs[p]
    pltpu.async_copy(sidx_hbm.at[pl.ds(b0, K)], sb, isems[p])
    pltpu.async_copy(gidx_hbm.at[pl.ds(b0, K)], gb, isems[p])
    pltpu.async_copy(ev_hbm.at[pl.ds(b0, K)], vb, isems[p])

  def wait_idx(p):
    sb, gb, vb = ibufs[p]
    pltpu.make_async_copy(sidx_hbm.at[pl.ds(0, K)], sb, isems[p]).wait()
    pltpu.make_async_copy(gidx_hbm.at[pl.ds(0, K)], gb, isems[p]).wait()
    pltpu.make_async_copy(ev_hbm.at[pl.ds(0, K)], vb, isems[p]).wait()

  def process(p):
    sidxb, gidxb, valb = ibufs[p]
    gd = {}
    sd = {}
    s_waited = [False] * K

    def fire_gather(k):
      gd[k] = pltpu.async_copy(
          tab_hbm.at[gidxb.at[k]], rows[k % NB], gsems[k % NB])

    for k in range(min(LA, K)):
      fire_gather(k)

    for k in range(K):
      gd[k].wait()

      @pl.loop(0, CH // 16)
      def _(g):
        vv = valb[k, pl.ds(g * 16, 16)]
        for j in range(16):
          v = vv[j]
          e = g * 16 + j
          rows[k % NB][e, pl.ds(0, 16)] = rows[k % NB][e, pl.ds(0, 16)] * v
          rows[k % NB][e, pl.ds(16, 16)] = rows[k % NB][e, pl.ds(16, 16)] * v

      sd[k] = pltpu.async_copy(
          rows[k % NB], acc.at[sidxb.at[k]], ssems[k % NB], add=True)
      nk = k + LA
      if nk < K:
        if nk >= NB:
          sd[nk - NB].wait()
          s_waited[nk - NB] = True
        fire_gather(nk)

    for k in range(K):
      if not s_waited[k]:
        sd[k].wait()

  fire_idx(s, 0)

  @pl.loop(0, NSLOT // 2)
  def _(bj):
    for ph in (0, 1):
      t = (2 * bj + ph) * NS + s
      tn = t + NS

      @pl.when(tn < NBLK)
      def _():
        fire_idx(tn, 1 - ph)

      @pl.when(t < NBLK)
      def _():
        wait_idx(ph)
        process(ph)


def _readback_l1(s, acc, base_hbm, d_hbm, out_hbm, abuf, bbuf, dbuf, obuf,
                 sems):
  def do_chunk(r0, n):
    da = pltpu.async_copy(acc.at[pl.ds(r0, n)], abuf.at[pl.ds(0, n)], sems[0])
    db = pltpu.async_copy(base_hbm.at[pl.ds(r0, n)], bbuf.at[pl.ds(0, n)],
                          sems[1])
    dd = pltpu.async_copy(d_hbm.at[pl.ds(r0, n)], dbuf.at[pl.ds(0, n)],
                          sems[2])
    da.wait()
    db.wait()
    dd.wait()

    @pl.loop(0, n // 16)
    def _(g):
      dv = dbuf[pl.ds(g * 16, 16)]
      for j in range(16):
        d = dv[j]
        e = g * 16 + j
        for h in (0, 16):
          obuf[e, pl.ds(h, 16)] = jnp.maximum(
              abuf[e, pl.ds(h, 16)] + bbuf[e, pl.ds(h, 16)] * d, 0.0)

    pltpu.sync_copy(obuf.at[pl.ds(0, n)], out_hbm.at[pl.ds(r0, n)])

  @pl.loop(0, RB_ITERS)
  def _(ti):
    t = ti * NS + s

    @pl.when(t < NODE_CHUNKS)
    def _():
      do_chunk(t * CH, CH)

    @pl.when(t == NODE_CHUNKS)
    def _():
      do_chunk(NODE_CHUNKS * CH, NTAIL)


def _readback_l2(s, acc, g1_hbm, emb_hbm, d_hbm, out_hbm,
                 abuf, bbuf, ebuf, dbuf, obuf, wbuf, ssqv, sems):
  wv = wbuf[...]
  w0 = wv[0]
  w1 = wv[1]
  w2 = wv[2]
  ssqv[...] = jnp.zeros((16,), _f32)

  def do_chunk(r0, n):
    da = pltpu.async_copy(acc.at[pl.ds(r0, n)], abuf.at[pl.ds(0, n)], sems[0])
    db = pltpu.async_copy(g1_hbm.at[pl.ds(r0, n)], bbuf.at[pl.ds(0, n)],
                          sems[1])
    de = pltpu.async_copy(emb_hbm.at[pl.ds(r0, n)], ebuf.at[pl.ds(0, n)],
                          sems[2])
    dd = pltpu.async_copy(d_hbm.at[pl.ds(r0, n)], dbuf.at[pl.ds(0, n)],
                          sems[3])
    da.wait()
    db.wait()
    de.wait()
    dd.wait()

    @pl.loop(0, n // 16)
    def _(g):
      dv = dbuf[pl.ds(g * 16, 16)]
      acc_sq = jnp.zeros((16,), _f32)
      for j in range(16):
        d = dv[j]
        e = g * 16 + j
        for h in (0, 16):
          g1 = bbuf[e, pl.ds(h, 16)]
          g2 = jnp.maximum(abuf[e, pl.ds(h, 16)] + g1 * d, 0.0)
          o = w0 * ebuf[e, pl.ds(h, 16)] + w1 * g1 + w2 * g2
          obuf[e, pl.ds(h, 16)] = o
          acc_sq = acc_sq + o * o
      ssqv[...] = ssqv[...] + acc_sq

    pltpu.sync_copy(obuf.at[pl.ds(0, n)], out_hbm.at[pl.ds(r0, n)])

  @pl.loop(0, RB_ITERS)
  def _(ti):
    t = ti * NS + s

    @pl.when(t < NODE_CHUNKS)
    def _():
      do_chunk(t * CH, CH)

    @pl.when(t == NODE_CHUNKS)
    def _():
      do_chunk(NODE_CHUNKS * CH, NTAIL)


def _batch_gather(s, idx_hbm, tab, btab, rows_out, b_out,
                  idxb, rowsb, sbuf, sems):
  @pl.loop(0, BG_CH)
  def _(j):
    base = (s * BG_CH + j) * CH
    pltpu.sync_copy(idx_hbm.at[pl.ds(base, CH)], idxb)
    dr = pltpu.async_copy(tab.at[idxb], rowsb, sems[0])
    db = pltpu.async_copy(btab.at[idxb], sbuf, sems[1])
    dr.wait()
    db.wait()
    pltpu.sync_copy(rowsb, rows_out.at[pl.ds(base, CH)])
    pltpu.sync_copy(sbuf, b_out.at[pl.ds(base, CH)])


def _layer1_body(eu, ei, ev, tu, ti, du, dv, out_u, out_i,
                 acc, sA, gA, vA, sB, gB, vB, r0, r1, r2, r3, r4, dbuf,
                 g0, g1, g2, g3, g4, s0, s1, s2, s3, s4, ia, ib):
  rows = [r0, r1, r2, r3, r4]
  gsems = [g0, g1, g2, g3, g4]
  ssems = [s0, s1, s2, s3, s4]
  isems = [ia, ib]
  ibufs = [(sA, gA, vA), (sB, gB, vB)]
  abuf, bbuf, obuf = r0, r1, r2
  c = lax.axis_index("c")
  s = lax.axis_index("s")
  _zero_acc(acc, obuf, s, ssems)
  plsc.subcore_barrier()

  @pl.when(c == 0)
  def _():
    _edge_phase(s, eu, ei, ev, ti, acc, ibufs, rows, gsems, ssems, isems)

  @pl.when(c == 1)
  def _():
    _edge_phase(s, ei, eu, ev, tu, acc, ibufs, rows, gsems, ssems, isems)

  plsc.subcore_barrier()

  @pl.when(c == 0)
  def _():
    _readback_l1(s, acc, tu, du, out_u, abuf, bbuf, dbuf, obuf, gsems)

  @pl.when(c == 1)
  def _():
    _readback_l1(s, acc, ti, dv, out_i, abuf, bbuf, dbuf, obuf, gsems)


def _layer2_body(eu, ei, ev, g1u, g1i, embu, embi, du, dv, wpad,
                 u0, i0, ubt, ibt,
                 out_u, out_i, parts, ug, ig, ubg, ibg,
                 acc, sA, gA, vA, sB, gB, vB, r0, r1, r2, r3, r4,
                 dbuf, wbuf, ssqv, sbuf,
                 g0, g1_, g2, g3, g4, s0, s1, s2, s3, s4, ia, ib):
  rows = [r0, r1, r2, r3, r4]
  gsems = [g0, g1_, g2, g3, g4]
  ssems = [s0, s1, s2, s3, s4]
  isems = [ia, ib]
  ibufs = [(sA, gA, vA), (sB, gB, vB)]
  abuf, bbuf, ebuf, obuf = r0, r1, r2, r3
  c = lax.axis_index("c")
  s = lax.axis_index("s")
  pltpu.sync_copy(wpad, wbuf)
  _zero_acc(acc, obuf, s, ssems)
  plsc.subcore_barrier()

  @pl.when(c == 0)
  def _():
    _edge_phase(s, eu, ei, ev, g1i, acc, ibufs, rows, gsems, ssems, isems)

  @pl.when(c == 1)
  def _():
    _edge_phase(s, ei, eu, ev, g1u, acc, ibufs, rows, gsems, ssems, isems)

  plsc.subcore_barrier()

  @pl.when(c == 0)
  def _():
    _readback_l2(s, acc, g1u, embu, du, out_u,
                 abuf, bbuf, ebuf, dbuf, obuf, wbuf, ssqv, gsems)

  @pl.when(c == 1)
  def _():
    _readback_l2(s, acc, g1i, embi, dv, out_i,
                 abuf, bbuf, ebuf, dbuf, obuf, wbuf, ssqv, gsems)

  pltpu.sync_copy(ssqv, parts.at[c, s])
  plsc.subcore_barrier()

  @pl.when(c == 0)
  def _():
    _batch_gather(s, u0, out_u, ubt, ug, ubg, gA.at[0], r4, sbuf, gsems)

  @pl.when(c == 1)
  def _():
    _batch_gather(s, i0, out_i, ibt, ig, ibg, gA.at[0], r4, sbuf, gsems)


_sc_params = pltpu.CompilerParams(use_tc_tiling_on_sc=False)

_layer1 = pl.kernel(
    _layer1_body,
    out_type=[jax.ShapeDtypeStruct((N, D), _f32)] * 2,
    mesh=_mesh,
    compiler_params=_sc_params,
    scratch_types=(
        [pltpu.VMEM_SHARED((N, D), _f32)]
        + [pltpu.VMEM((K, CH), _i32), pltpu.VMEM((K, CH), _i32),
           pltpu.VMEM((K, CH), _f32)] * 2
        + [pltpu.VMEM((CH, D), _f32)] * NB
        + [pltpu.VMEM((CH,), _f32)]
        + [pltpu.SemaphoreType.DMA] * (2 * NB + 2)
    ),
)

_layer2 = pl.kernel(
    _layer2_body,
    out_type=[
        jax.ShapeDtypeStruct((N, D), _f32),
        jax.ShapeDtypeStruct((N, D), _f32),
        jax.ShapeDtypeStruct((NC, NS, 16), _f32),
        jax.ShapeDtypeStruct((B, D), _f32),
        jax.ShapeDtypeStruct((B, D), _f32),
        jax.ShapeDtypeStruct((B, 1), _f32),
        jax.ShapeDtypeStruct((B, 1), _f32),
    ],
    mesh=_mesh,
    compiler_params=_sc_params,
    scratch_types=(
        [pltpu.VMEM_SHARED((N, D), _f32)]
        + [pltpu.VMEM((K, CH), _i32), pltpu.VMEM((K, CH), _i32),
           pltpu.VMEM((K, CH), _f32)] * 2
        + [pltpu.VMEM((CH, D), _f32)] * NB
        + [pltpu.VMEM((CH,), _f32), pltpu.VMEM((16,), _f32),
           pltpu.VMEM((16,), _f32), pltpu.VMEM((CH, 1), _f32)]
        + [pltpu.SemaphoreType.DMA] * (2 * NB + 2)
    ),
)

_BLK = 2048
_NB_TC = B // _BLK


def _leaky(x):
  return jnp.where(x >= 0, x, 0.1 * x)


def _tc_body(ug, ig, ubg, ibg, rat, w1t, b1r, w2t, b2r, avg, parts,
             loss_ref, loss2_ref):
  i = pl.program_id(0)

  def filt(x):
    h = _leaky(jnp.dot(x, w1t[...], preferred_element_type=_f32) + b1r[...])
    return _leaky(jnp.dot(h, w2t[...], preferred_element_type=_f32) + b2r[...])

  u = filt(ug[...])
  v = filt(ig[...])
  pred = (jnp.sum(u * v, axis=1, keepdims=True) + ubg[...] + ibg[...]
          + avg[0, 0])
  sq = jnp.sum((pred - rat[...]) ** 2)

  @pl.when(i == 0)
  def _():
    loss2_ref[...] = jnp.zeros((1, 1), _f32)

  loss2_ref[...] = loss2_ref[...] + sq

  @pl.when(i == _NB_TC - 1)
  def _():
    l2 = LAM * jnp.sum(parts[...]) / (N * D)
    m = loss2_ref[...] / B
    loss2_ref[...] = m
    loss_ref[...] = m + l2


_tck = pl.pallas_call(
    _tc_body,
    grid=(_NB_TC,),
    in_specs=[
        pl.BlockSpec((_BLK, D), lambda i: (i, 0)),
        pl.BlockSpec((_BLK, D), lambda i: (i, 0)),
        pl.BlockSpec((_BLK, 1), lambda i: (i, 0)),
        pl.BlockSpec((_BLK, 1), lambda i: (i, 0)),
        pl.BlockSpec((_BLK, 1), lambda i: (i, 0)),
        pl.BlockSpec((D, 2 * D), lambda i: (0, 0)),
        pl.BlockSpec((1, 2 * D), lambda i: (0, 0)),
        pl.BlockSpec((2 * D, D), lambda i: (0, 0)),
        pl.BlockSpec((1, D), lambda i: (0, 0)),
        pl.BlockSpec((1, 1), lambda i: (0, 0)),
        pl.BlockSpec((NC * NS, 16), lambda i: (0, 0)),
    ],
    out_specs=[
        pl.BlockSpec((1, 1), lambda i: (0, 0)),
        pl.BlockSpec((1, 1), lambda i: (0, 0)),
    ],
    out_shape=[
        jax.ShapeDtypeStruct((1, 1), _f32),
        jax.ShapeDtypeStruct((1, 1), _f32),
    ],
)


@jax.jit
def kernel(user0, item_i0, ratings, embed_user, embed_item, edge_u, edge_i,
           edge_val, d_i, d_j, W_add, W1, b1, W2, b2, ub_tab, ib_tab,
           avg_rating):
  eu2 = jnp.reshape(edge_u.astype(_i32), (ECH, CH))
  ei2 = jnp.reshape(edge_i.astype(_i32), (ECH, CH))
  ev2 = jnp.reshape(edge_val.astype(_f32), (ECH, CH))
  u0 = user0.astype(_i32)
  it0 = item_i0.astype(_i32)
  dif = jnp.reshape(d_i, (N,)).astype(_f32)
  djf = jnp.reshape(d_j, (N,)).astype(_f32)

  g1u, g1i = _layer1(eu2, ei2, ev2, embed_user, embed_item, dif, djf)
  wpad = jnp.zeros((16,), _f32).at[0:3].set(W_add[0].astype(_f32))
  gu, gi, parts, ug, ig, ubg, ibg = _layer2(
      eu2, ei2, ev2, g1u, g1i, embed_user, embed_item, dif, djf, wpad,
      u0, it0, ub_tab, ib_tab)
  loss, loss2 = _tck(ug, ig, ubg, ibg, jnp.reshape(ratings, (B, 1)),
                     W1.T, jnp.reshape(b1, (1, 2 * D)),
                     W2.T, jnp.reshape(b2, (1, D)),
                     jnp.reshape(avg_rating, (1, 1)).astype(_f32),
                     jnp.reshape(parts, (NC * NS, 16)))
  return (loss[0, 0], loss2[0, 0])


# speedup vs baseline: 29.0268x; 1.0445x over previous
import jax
import jax.numpy as jnp
from jax import lax
from jax.experimental import pallas as pl
from jax.experimental.pallas import tpu as pltpu
from jax.experimental.pallas import tpu_sc as plsc

N = 50000
D = 32
E = 1600000
B = 16384
LAM = 0.001
NC = 2
NS = 16
CH = 128
ECH = E // CH
K = 10
NB = 5
LA = 3
NBLK = ECH // K
NSLOT = ((NBLK + NS - 1) // NS + 1) // 2 * 2
NODE_CHUNKS = N // CH
NTAIL = N - NODE_CHUNKS * CH
RB_ITERS = NODE_CHUNKS // NS + 1
BG_CH = B // NS // CH

_mesh = plsc.VectorSubcoreMesh(
    core_axis_name="c", subcore_axis_name="s", num_cores=NC, num_subcores=NS)

_f32 = jnp.float32
_i32 = jnp.int32


def _zero_rows(buf, n):
  z = jnp.zeros((16,), _f32)

  @pl.loop(0, n)
  def _(e):
    buf[e, pl.ds(0, 16)] = z
    buf[e, pl.ds(16, 16)] = z


def _zero_acc(acc, zbuf, s, sems):
  _zero_rows(zbuf, CH)
  nw = len(sems)

  def fires(ti):
    t = ti * NS + s

    @pl.when(t < NODE_CHUNKS)
    def _():
      pltpu.async_copy(zbuf, acc.at[pl.ds(t * CH, CH)], sems[ti % nw])

    @pl.when(t == NODE_CHUNKS)
    def _():
      pltpu.async_copy(zbuf.at[pl.ds(0, NTAIL)],
                       acc.at[pl.ds(NODE_CHUNKS * CH, NTAIL)], sems[ti % nw])

  def waits(ti):
    t = ti * NS + s

    @pl.when(t < NODE_CHUNKS)
    def _():
      pltpu.make_async_copy(zbuf, acc.at[pl.ds(t * CH, CH)],
                            sems[ti % nw]).wait()

    @pl.when(t == NODE_CHUNKS)
    def _():
      pltpu.make_async_copy(zbuf.at[pl.ds(0, NTAIL)],
                            acc.at[pl.ds(NODE_CHUNKS * CH, NTAIL)],
                            sems[ti % nw]).wait()

  for w0 in range(0, RB_ITERS, 5):
    wave = range(w0, min(w0 + 5, RB_ITERS))
    for ti in wave:
      fires(ti)
    for ti in wave:
      waits(ti)


def _edge_phase(s, sidx_hbm, gidx_hbm, ev_hbm, tab_hbm, acc,
                ibufs, rows, gsems, ssems, isems):
  def fire_idx(t, p):
    b0 = t * K
    sb, gb, vb = ibufs[p]
    pltpu.async_copy(sidx_hbm.at[pl.ds(b0, K)], sb, isems[p])
    pltpu.async_copy(gidx_hbm.at[pl.ds(b0, K)], gb, isems[p])
    pltpu.async_copy(ev_hbm.at[pl.ds(b0, K)], vb, isems[p])

  def wait_idx(p):
    sb, gb, vb = ibufs[p]
    pltpu.make_async_copy(sidx_hbm.at[pl.ds(0, K)], sb, isems[p]).wait()
    pltpu.make_async_copy(gidx_hbm.at[pl.ds(0, K)], gb, isems[p]).wait()
    pltpu.make_async_copy(ev_hbm.at[pl.ds(0, K)], vb, isems[p]).wait()

  def process(p):
    sidxb, gidxb, valb = ibufs[p]
    gd = {}
    sd = {}
    s_waited = [False] * K

    def fire_gather(k):
      gd[k] = pltpu.async_copy(
          tab_hbm.at[gidxb.at[k]], rows[k % NB], gsems[k % NB])

    for k in range(min(LA, K)):
      fire_gather(k)

    for k in range(K):
      gd[k].wait()

      @pl.loop(0, CH // 16)
      def _(g):
        vv = valb[k, pl.ds(g * 16, 16)]
        for j in range(16):
          v = vv[j]
          e = g * 16 + j
          rows[k % NB][e, pl.ds(0, 16)] = rows[k % NB][e, pl.ds(0, 16)] * v
          rows[k % NB][e, pl.ds(16, 16)] = rows[k % NB][e, pl.ds(16, 16)] * v

      sd[k] = pltpu.async_copy(
          rows[k % NB], acc.at[sidxb.at[k]], ssems[k % NB], add=True)
      nk = k + LA
      if nk < K:
        if nk >= NB:
          sd[nk - NB].wait()
          s_waited[nk - NB] = True
        fire_gather(nk)

    for k in range(K):
      if not s_waited[k]:
        sd[k].wait()

  fire_idx(s, 0)

  @pl.loop(0, NSLOT // 2)
  def _(bj):
    for ph in (0, 1):
      t = (2 * bj + ph) * NS + s
      tn = t + NS

      @pl.when(tn < NBLK)
      def _():
        fire_idx(tn, 1 - ph)

      @pl.when(t < NBLK)
      def _():
        wait_idx(ph)
        process(ph)


def _readback_l1(s, acc, base_hbm, d_hbm, out_hbm, abuf, bbuf, dbuf, obuf,
                 sems):
  def do_chunk(r0, n):
    da = pltpu.async_copy(acc.at[pl.ds(r0, n)], abuf.at[pl.ds(0, n)], sems[0])
    db = pltpu.async_copy(base_hbm.at[pl.ds(r0, n)], bbuf.at[pl.ds(0, n)],
                          sems[1])
    dd = pltpu.async_copy(d_hbm.at[pl.ds(r0, n)], dbuf.at[pl.ds(0, n)],
                          sems[2])
    da.wait()
    db.wait()
    dd.wait()

    @pl.loop(0, n // 16)
    def _(g):
      dv = dbuf[pl.ds(g * 16, 16)]
      for j in range(16):
        d = dv[j]
        e = g * 16 + j
        for h in (0, 16):
          obuf[e, pl.ds(h, 16)] = jnp.maximum(
              abuf[e, pl.ds(h, 16)] + bbuf[e, pl.ds(h, 16)] * d, 0.0)

    pltpu.sync_copy(obuf.at[pl.ds(0, n)], out_hbm.at[pl.ds(r0, n)])

  @pl.loop(0, RB_ITERS)
  def _(ti):
    t = ti * NS + s

    @pl.when(t < NODE_CHUNKS)
    def _():
      do_chunk(t * CH, CH)

    @pl.when(t == NODE_CHUNKS)
    def _():
      do_chunk(NODE_CHUNKS * CH, NTAIL)


def _readback_l2(s, acc, g1_hbm, emb_hbm, d_hbm,
                 abuf, bbuf, ebuf, dbuf, obuf, wbuf, ssqv, sems):
  wv = wbuf[...]
  w0 = wv[0]
  w1 = wv[1]
  w2 = wv[2]
  ssqv[...] = jnp.zeros((16,), _f32)

  def do_chunk(r0, n):
    da = pltpu.async_copy(acc.at[pl.ds(r0, n)], abuf.at[pl.ds(0, n)], sems[0])
    db = pltpu.async_copy(g1_hbm.at[pl.ds(r0, n)], bbuf.at[pl.ds(0, n)],
                          sems[1])
    de = pltpu.async_copy(emb_hbm.at[pl.ds(r0, n)], ebuf.at[pl.ds(0, n)],
                          sems[2])
    dd = pltpu.async_copy(d_hbm.at[pl.ds(r0, n)], dbuf.at[pl.ds(0, n)],
                          sems[3])
    da.wait()
    db.wait()
    de.wait()
    dd.wait()

    @pl.loop(0, n // 16)
    def _(g):
      dv = dbuf[pl.ds(g * 16, 16)]
      acc_sq = jnp.zeros((16,), _f32)
      for j in range(16):
        d = dv[j]
        e = g * 16 + j
        for h in (0, 16):
          g1 = bbuf[e, pl.ds(h, 16)]
          g2 = jnp.maximum(abuf[e, pl.ds(h, 16)] + g1 * d, 0.0)
          o = w0 * ebuf[e, pl.ds(h, 16)] + w1 * g1 + w2 * g2
          obuf[e, pl.ds(h, 16)] = o
          acc_sq = acc_sq + o * o
      ssqv[...] = ssqv[...] + acc_sq

    pltpu.sync_copy(obuf.at[pl.ds(0, n)], acc.at[pl.ds(r0, n)])

  @pl.loop(0, RB_ITERS)
  def _(ti):
    t = ti * NS + s

    @pl.when(t < NODE_CHUNKS)
    def _():
      do_chunk(t * CH, CH)

    @pl.when(t == NODE_CHUNKS)
    def _():
      do_chunk(NODE_CHUNKS * CH, NTAIL)


def _batch_gather(s, idx_hbm, tab_sp, btab, rows_out, b_out,
                  idxbufs, rbufs, bbufs, sems):
  idxd = {}
  stored = {}
  for j in range(BG_CH):
    p = j % 2
    base = (s * BG_CH + j) * CH
    if j == 0:
      pltpu.sync_copy(idx_hbm.at[pl.ds(base, CH)], idxbufs[0])
    else:
      idxd[j].wait()
    if j + 1 < BG_CH:
      idxd[j + 1] = pltpu.async_copy(
          idx_hbm.at[pl.ds(base + CH, CH)], idxbufs[1 - p], sems[2])
    if j >= 2:
      for dsc in stored[j - 2]:
        dsc.wait()
    dr = pltpu.async_copy(tab_sp.at[idxbufs[p]], rbufs[p], sems[0])
    db = pltpu.async_copy(btab.at[idxbufs[p]], bbufs[p], sems[1])
    dr.wait()
    db.wait()
    stored[j] = (
        pltpu.async_copy(rbufs[p], rows_out.at[pl.ds(base, CH)], sems[3]),
        pltpu.async_copy(bbufs[p], b_out.at[pl.ds(base, CH)], sems[4]),
    )
  for j in (BG_CH - 2, BG_CH - 1):
    for dsc in stored[j]:
      dsc.wait()


def _layer1_body(eu, ei, ev, tu, ti, du, dv, out_u, out_i,
                 acc, sA, gA, vA, sB, gB, vB, r0, r1, r2, r3, r4, dbuf,
                 g0, g1, g2, g3, g4, s0, s1, s2, s3, s4, ia, ib):
  rows = [r0, r1, r2, r3, r4]
  gsems = [g0, g1, g2, g3, g4]
  ssems = [s0, s1, s2, s3, s4]
  isems = [ia, ib]
  ibufs = [(sA, gA, vA), (sB, gB, vB)]
  abuf, bbuf, obuf = r0, r1, r2
  c = lax.axis_index("c")
  s = lax.axis_index("s")
  _zero_acc(acc, obuf, s, ssems)
  plsc.subcore_barrier()

  @pl.when(c == 0)
  def _():
    _edge_phase(s, eu, ei, ev, ti, acc, ibufs, rows, gsems, ssems, isems)

  @pl.when(c == 1)
  def _():
    _edge_phase(s, ei, eu, ev, tu, acc, ibufs, rows, gsems, ssems, isems)

  plsc.subcore_barrier()

  @pl.when(c == 0)
  def _():
    _readback_l1(s, acc, tu, du, out_u, abuf, bbuf, dbuf, obuf, gsems)

  @pl.when(c == 1)
  def _():
    _readback_l1(s, acc, ti, dv, out_i, abuf, bbuf, dbuf, obuf, gsems)


def _layer2_body(eu, ei, ev, g1u, g1i, embu, embi, du, dv, wpad,
                 u0, i0, ubt, ibt,
                 parts, ug, ig, ubg, ibg,
                 acc, sA, gA, vA, sB, gB, vB, r0, r1, r2, r3, r4,
                 dbuf, wbuf, ssqv,
                 g0, g1_, g2, g3, g4, s0, s1, s2, s3, s4, ia, ib):
  rows = [r0, r1, r2, r3, r4]
  gsems = [g0, g1_, g2, g3, g4]
  ssems = [s0, s1, s2, s3, s4]
  isems = [ia, ib]
  ibufs = [(sA, gA, vA), (sB, gB, vB)]
  abuf, bbuf, ebuf, obuf = r0, r1, r2, r3
  c = lax.axis_index("c")
  s = lax.axis_index("s")
  pltpu.sync_copy(wpad, wbuf)
  _zero_acc(acc, obuf, s, ssems)
  plsc.subcore_barrier()

  @pl.when(c == 0)
  def _():
    _edge_phase(s, eu, ei, ev, g1i, acc, ibufs, rows, gsems, ssems, isems)

  @pl.when(c == 1)
  def _():
    _edge_phase(s, ei, eu, ev, g1u, acc, ibufs, rows, gsems, ssems, isems)

  plsc.subcore_barrier()

  @pl.when(c == 0)
  def _():
    _readback_l2(s, acc, g1u, embu, du,
                 abuf, bbuf, ebuf, dbuf, obuf, wbuf, ssqv, gsems)

  @pl.when(c == 1)
  def _():
    _readback_l2(s, acc, g1i, embi, dv,
                 abuf, bbuf, ebuf, dbuf, obuf, wbuf, ssqv, gsems)

  pltpu.sync_copy(ssqv, parts.at[c, s])
  plsc.subcore_barrier()

  idxbufs = [gA.at[0], gB.at[0]]
  rbufs = [r4, r3]
  bbufs = [dbuf, vA.at[0]]

  @pl.when(c == 0)
  def _():
    _batch_gather(s, u0, acc, ubt, ug, ubg, idxbufs, rbufs, bbufs, gsems)

  @pl.when(c == 1)
  def _():
    _batch_gather(s, i0, acc, ibt, ig, ibg, idxbufs, rbufs, bbufs, gsems)


_sc_params = pltpu.CompilerParams(use_tc_tiling_on_sc=False)

_layer1 = pl.kernel(
    _layer1_body,
    out_type=[jax.ShapeDtypeStruct((N, D), _f32)] * 2,
    mesh=_mesh,
    compiler_params=_sc_params,
    scratch_types=(
        [pltpu.VMEM_SHARED((N, D), _f32)]
        + [pltpu.VMEM((K, CH), _i32), pltpu.VMEM((K, CH), _i32),
           pltpu.VMEM((K, CH), _f32)] * 2
        + [pltpu.VMEM((CH, D), _f32)] * NB
        + [pltpu.VMEM((CH,), _f32)]
        + [pltpu.SemaphoreType.DMA] * (2 * NB + 2)
    ),
)

_layer2 = pl.kernel(
    _layer2_body,
    out_type=[
        jax.ShapeDtypeStruct((NC, NS, 16), _f32),
        jax.ShapeDtypeStruct((B, D), _f32),
        jax.ShapeDtypeStruct((B, D), _f32),
        jax.ShapeDtypeStruct((B,), _f32),
        jax.ShapeDtypeStruct((B,), _f32),
    ],
    mesh=_mesh,
    compiler_params=_sc_params,
    scratch_types=(
        [pltpu.VMEM_SHARED((N, D), _f32)]
        + [pltpu.VMEM((K, CH), _i32), pltpu.VMEM((K, CH), _i32),
           pltpu.VMEM((K, CH), _f32)] * 2
        + [pltpu.VMEM((CH, D), _f32)] * NB
        + [pltpu.VMEM((CH,), _f32), pltpu.VMEM((16,), _f32),
           pltpu.VMEM((16,), _f32)]
        + [pltpu.SemaphoreType.DMA] * (2 * NB + 2)
    ),
)

_BLK = 2048
_NB_TC = B // _BLK


def _leaky(x):
  return jnp.where(x >= 0, x, 0.1 * x)


def _tc_body(ug, ig, ubg, ibg, rat, w1t, b1r, w2t, b2r, avg, parts,
             loss_ref, loss2_ref):
  i = pl.program_id(0)

  def filt(x):
    h = _leaky(jnp.dot(x, w1t[...], preferred_element_type=_f32) + b1r[...])
    return _leaky(jnp.dot(h, w2t[...], preferred_element_type=_f32) + b2r[...])

  u = filt(ug[...])
  v = filt(ig[...])
  pred = (jnp.sum(u * v, axis=1, keepdims=True) + ubg[...] + ibg[...]
          + avg[0, 0])
  sq = jnp.sum((pred - rat[...]) ** 2)

  @pl.when(i == 0)
  def _():
    loss2_ref[...] = jnp.zeros((1, 1), _f32)

  loss2_ref[...] = loss2_ref[...] + sq

  @pl.when(i == _NB_TC - 1)
  def _():
    l2 = LAM * jnp.sum(parts[...]) / (N * D)
    m = loss2_ref[...] / B
    loss2_ref[...] = m
    loss_ref[...] = m + l2


_tck = pl.pallas_call(
    _tc_body,
    grid=(_NB_TC,),
    in_specs=[
        pl.BlockSpec((_BLK, D), lambda i: (i, 0)),
        pl.BlockSpec((_BLK, D), lambda i: (i, 0)),
        pl.BlockSpec((_BLK, 1), lambda i: (i, 0)),
        pl.BlockSpec((_BLK, 1), lambda i: (i, 0)),
        pl.BlockSpec((_BLK, 1), lambda i: (i, 0)),
        pl.BlockSpec((D, 2 * D), lambda i: (0, 0)),
        pl.BlockSpec((1, 2 * D), lambda i: (0, 0)),
        pl.BlockSpec((2 * D, D), lambda i: (0, 0)),
        pl.BlockSpec((1, D), lambda i: (0, 0)),
        pl.BlockSpec((1, 1), lambda i: (0, 0)),
        pl.BlockSpec((NC * NS, 16), lambda i: (0, 0)),
    ],
    out_specs=[
        pl.BlockSpec((1, 1), lambda i: (0, 0)),
        pl.BlockSpec((1, 1), lambda i: (0, 0)),
    ],
    out_shape=[
        jax.ShapeDtypeStruct((1, 1), _f32),
        jax.ShapeDtypeStruct((1, 1), _f32),
    ],
)


@jax.jit
def kernel(user0, item_i0, ratings, embed_user, embed_item, edge_u, edge_i,
           edge_val, d_i, d_j, W_add, W1, b1, W2, b2, ub_tab, ib_tab,
           avg_rating):
  eu2 = jnp.reshape(edge_u.astype(_i32), (ECH, CH))
  ei2 = jnp.reshape(edge_i.astype(_i32), (ECH, CH))
  ev2 = jnp.reshape(edge_val.astype(_f32), (ECH, CH))
  u0 = user0.astype(_i32)
  it0 = item_i0.astype(_i32)
  dif = jnp.reshape(d_i, (N,)).astype(_f32)
  djf = jnp.reshape(d_j, (N,)).astype(_f32)

  g1u, g1i = _layer1(eu2, ei2, ev2, embed_user, embed_item, dif, djf)
  wpad = jnp.zeros((16,), _f32).at[0:3].set(W_add[0].astype(_f32))
  parts, ug, ig, ubg, ibg = _layer2(
      eu2, ei2, ev2, g1u, g1i, embed_user, embed_item, dif, djf, wpad,
      u0, it0, jnp.reshape(ub_tab, (N,)), jnp.reshape(ib_tab, (N,)))
  loss, loss2 = _tck(ug, ig, jnp.reshape(ubg, (B, 1)),
                     jnp.reshape(ibg, (B, 1)), jnp.reshape(ratings, (B, 1)),
                     W1.T, jnp.reshape(b1, (1, 2 * D)),
                     W2.T, jnp.reshape(b2, (1, D)),
                     jnp.reshape(avg_rating, (1, 1)).astype(_f32),
                     jnp.reshape(parts, (NC * NS, 16)))
  return (loss[0, 0], loss2[0, 0])
